# baseline (speedup 1.0000x reference)
import functools

import jax
import jax.numpy as jnp
import numpy as np
from jax import lax
from jax.experimental import pallas as pl
from jax.experimental.pallas import tpu as pltpu
from jax.experimental.pallas import tpu_sc as plsc

VOCAB = 20000
EMBED = 50
HIDDEN = 100
BATCH = 4096
SEQLEN = 200
BN_EPS = 1e-5

NC = 2
NS = 16
LANES = 16
NW = NC * NS
BPW = BATCH // NW
CH0 = 104
CH1 = SEQLEN - CH0
NBUF = 4
EPAD = 64
HPAD = 128

_PERM = np.concatenate([
    np.arange(0, 32, 2), np.arange(1, 32, 2),
    np.arange(32, 64, 2), np.arange(33, 64, 2),
])


def _sc_pool(table, x1d):
    mesh = plsc.VectorSubcoreMesh(core_axis_name="c", subcore_axis_name="s")

    @functools.partial(
        pl.kernel,
        out_type=jax.ShapeDtypeStruct((BATCH // 2, 2 * EPAD), jnp.float32),
        mesh=mesh,
        scratch_types=[
            pltpu.VMEM((BPW * SEQLEN,), jnp.int32),
            *[pltpu.VMEM((SEQLEN, EPAD), jnp.bfloat16) for _ in range(NBUF)],
            pltpu.VMEM((BPW // 2, 2 * EPAD), jnp.float32),
            *[pltpu.SemaphoreType.DMA for _ in range(NBUF)],
        ],
        compiler_params=pltpu.CompilerParams(
            needs_layout_passes=False, use_tc_tiling_on_sc=False
        ),
    )
    def pool(table_hbm, x_hbm, out_hbm, idx_v, *rest):
        bufs = rest[:NBUF]
        acc_v = rest[NBUF]
        sems = rest[NBUF + 1:]
        wid = lax.axis_index("s") * NC + lax.axis_index("c")

        pltpu.sync_copy(x_hbm.at[pl.ds(wid * BPW * SEQLEN, BPW * SEQLEN)], idx_v)

        def fire(e, b):
            pltpu.async_copy(
                table_hbm.at[idx_v.at[pl.ds(e * SEQLEN, CH0)]],
                bufs[b].at[pl.ds(0, CH0)],
                sems[b],
            )
            pltpu.async_copy(
                table_hbm.at[idx_v.at[pl.ds(e * SEQLEN + CH0, CH1)]],
                bufs[b].at[pl.ds(CH0, CH1)],
                sems[b],
            )

        for b in range(NBUF):
            fire(b, b)

        inv = jnp.float32(1.0 / SEQLEN)
        zero = jnp.zeros((LANES,), jnp.float32)

        def reduce_block(buf):
            def grp(g, carry):
                a0, a1, a2, a3 = carry
                r0 = g * 8
                for r in range(8):
                    c0 = buf[r0 + r, pl.ds(0, 2 * LANES)]
                    c1 = buf[r0 + r, pl.ds(2 * LANES, 2 * LANES)]
                    e0, o0 = plsc.unpack(c0, format=plsc.PackFormat.INTERLEAVED)
                    e1, o1 = plsc.unpack(c1, format=plsc.PackFormat.INTERLEAVED)
                    a0 = a0 + e0
                    a1 = a1 + o0
                    a2 = a2 + e1
                    a3 = a3 + o1
                return a0, a1, a2, a3

            return lax.fori_loop(0, SEQLEN // 8, grp, (zero, zero, zero, zero))

        def outer(i, _):
            for b in range(NBUF):
                e = i * NBUF + b
                pltpu.make_async_copy(
                    table_hbm.at[pl.ds(0, SEQLEN)], bufs[b], sems[b]
                ).wait()
                a0, a1, a2, a3 = reduce_block(bufs[b])

                @pl.when(e + NBUF < BPW)
                def _():
                    fire(e + NBUF, b)

                row = i * (NBUF // 2) + (b // 2)
                col0 = (b % 2) * EPAD
                acc_v[row, pl.ds(col0, LANES)] = a0 * inv
                acc_v[row, pl.ds(col0 + LANES, LANES)] = a1 * inv
                acc_v[row, pl.ds(col0 + 2 * LANES, LANES)] = a2 * inv
                acc_v[row, pl.ds(col0 + 3 * LANES, LANES)] = a3 * inv
            return 0

        lax.fori_loop(0, BPW // NBUF, outer, 0)
        pltpu.sync_copy(acc_v, out_hbm.at[pl.ds(wid * (BPW // 2), BPW // 2)])

    return pool(table, x1d)


def _mlp(pooled2, w1p, b1p, bns, bnb, bnm, bnv, w2row, b2p):
    def body(p_ref, w1_ref, b1_ref, s_ref, bb_ref, m_ref, v_ref, w2_ref, b2_ref, o_ref):
        h = p_ref[...]
        h = jnp.where(h >= 0, h, 0.01 * h)
        s = s_ref[...] * lax.rsqrt(v_ref[...] + BN_EPS)
        t = bb_ref[...] - m_ref[...] * s

        def head(hh):
            h1 = jnp.dot(hh, w1_ref[...], preferred_element_type=jnp.float32) + b1_ref[...]
            h1 = jnp.where(h1 >= 0, h1, 0.01 * h1)
            h1 = h1 * s + t
            return jnp.sum(h1 * w2_ref[...], axis=1, keepdims=True) + b2_ref[..., :1]

        o_ref[:, 0:1] = head(h[:, 0:EPAD])
        o_ref[:, 1:2] = head(h[:, EPAD:2 * EPAD])

    grid = 8
    bb = BATCH // 2 // grid
    vec_spec = pl.BlockSpec((1, HPAD), lambda i: (0, 0))
    return pl.pallas_call(
        body,
        grid=(grid,),
        in_specs=[
            pl.BlockSpec((bb, 2 * EPAD), lambda i: (i, 0)),
            pl.BlockSpec((EPAD, HPAD), lambda i: (0, 0)),
            vec_spec, vec_spec, vec_spec, vec_spec, vec_spec, vec_spec, vec_spec,
        ],
        out_specs=pl.BlockSpec((bb, 2), lambda i: (i, 0)),
        out_shape=jax.ShapeDtypeStruct((BATCH // 2, 2), jnp.float32),
    )(pooled2, w1p, b1p, bns, bnb, bnm, bnv, w2row, b2p)


def kernel(x, embed_table, W1, b1, bn_scale, bn_bias, bn_mean, bn_var, W2, b2):
    f32 = jnp.float32
    x1d = x.astype(jnp.int32).reshape(BATCH * SEQLEN)
    tpad = (
        jnp.zeros((VOCAB, EPAD), jnp.bfloat16)
        .at[:, :EMBED].set(embed_table.astype(jnp.bfloat16))
    )
    pooled2 = _sc_pool(tpad, x1d)

    w1p = jnp.zeros((EPAD, HPAD), f32).at[:EMBED, :HIDDEN].set(W1)
    w1p = w1p[jnp.asarray(_PERM), :]
    b1p = jnp.zeros((1, HPAD), f32).at[0, :HIDDEN].set(b1)
    bns = jnp.zeros((1, HPAD), f32).at[0, :HIDDEN].set(bn_scale)
    bnb = jnp.zeros((1, HPAD), f32).at[0, :HIDDEN].set(bn_bias)
    bnm = jnp.zeros((1, HPAD), f32).at[0, :HIDDEN].set(bn_mean)
    bnv = jnp.ones((1, HPAD), f32).at[0, :HIDDEN].set(bn_var)
    w2row = jnp.zeros((1, HPAD), f32).at[0, :HIDDEN].set(W2[:, 0])
    b2p = jnp.broadcast_to(b2.reshape(1, 1), (1, HPAD))

    out = _mlp(pooled2, w1p, b1p, bns, bnb, bnm, bnv, w2row, b2p)
    return out.reshape(BATCH)
